# baseline (speedup 1.0000x reference)
import functools
import math

import jax
import jax.numpy as jnp
from jax import lax
from jax.experimental import pallas as pl
from jax.experimental.pallas import tpu as pltpu
from jax.experimental.pallas import tpu_sc as plsc

_C1 = -2.0 * math.sqrt(2.0 / math.pi)
_C3 = -2.0 * math.sqrt(2.0 / math.pi) * 0.044715

_NC, _NS = 2, 16
_NW = _NC * _NS
_CH = 16384


def _sc_gelu_body(x_hbm, o_hbm, buf_in, buf_out):
    wid = lax.axis_index("c") * _NS + lax.axis_index("s")
    span = x_hbm.shape[0] // _NW
    base = wid * span

    def chunk(ci, carry):
        off = base + ci * _CH
        pltpu.sync_copy(x_hbm.at[pl.ds(off, _CH)], buf_in)

        def vec(vi, c):
            v = buf_in[pl.ds(vi * 16, 16)]
            u = v * v
            w = u * v
            e = jnp.exp(_C1 * v + _C3 * w)
            buf_out[pl.ds(vi * 16, 16)] = v / (1.0 + e)
            return c

        lax.fori_loop(0, _CH // 16, vec, 0, unroll=4)
        pltpu.sync_copy(buf_out, o_hbm.at[pl.ds(off, _CH)])
        return carry

    lax.fori_loop(0, span // _CH, chunk, 0)


def _sc_gelu(flat):
    n = flat.shape[0]
    mesh = plsc.VectorSubcoreMesh(core_axis_name="c", subcore_axis_name="s")
    return pl.kernel(
        _sc_gelu_body,
        out_type=jax.ShapeDtypeStruct((n,), jnp.float32),
        mesh=mesh,
        scratch_types=[
            pltpu.VMEM((_CH,), jnp.float32),
            pltpu.VMEM((_CH,), jnp.float32),
        ],
    )(flat)


def kernel(x, logit_decay, log_tau, log_blend):
    del logit_decay, log_tau, log_blend
    B, T, D = x.shape
    out = _sc_gelu(x.reshape(B * T * D))
    return out.reshape(B, T, D)


# speedup vs baseline: 22.1368x; 22.1368x over previous
import math

import jax
import jax.numpy as jnp
from jax.experimental import pallas as pl

_SQRT_2_OVER_PI = math.sqrt(2.0 / math.pi)


def _gelu_body(x_ref, o_ref):
    x = x_ref[...]
    inner = _SQRT_2_OVER_PI * (x + 0.044715 * (x * x * x))
    o_ref[...] = 0.5 * x * (1.0 + jnp.tanh(inner))


def kernel(x, logit_decay, log_tau, log_blend):
    del logit_decay, log_tau, log_blend
    B, T, D = x.shape
    x2 = x.reshape(B * T, D)
    bm = 1024
    grid = (x2.shape[0] // bm,)
    out = pl.pallas_call(
        _gelu_body,
        grid=grid,
        in_specs=[pl.BlockSpec((bm, D), lambda i: (i, 0))],
        out_specs=pl.BlockSpec((bm, D), lambda i: (i, 0)),
        out_shape=jax.ShapeDtypeStruct(x2.shape, x2.dtype),
    )(x2)
    return out.reshape(B, T, D)


# speedup vs baseline: 22.6258x; 1.0221x over previous
import math

import jax
import jax.numpy as jnp
from jax import lax
from jax.experimental import pallas as pl
from jax.experimental.pallas import tpu as pltpu

_SQRT_2_OVER_PI = math.sqrt(2.0 / math.pi)

_BM = 256
_NBUF = 6


def _gelu(x):
    inner = _SQRT_2_OVER_PI * (x + 0.044715 * (x * x * x))
    return 0.5 * x * (1.0 + jnp.tanh(inner))


def _pipe_body(x_hbm, o_hbm, ibuf, obuf, isem, osem):
    n = x_hbm.shape[0] // _BM

    for s in range(_NBUF):
        pltpu.make_async_copy(
            x_hbm.at[pl.ds(s * _BM, _BM), :], ibuf.at[s], isem.at[s]
        ).start()

    def step(i, carry):
        s = lax.rem(i, _NBUF)
        pltpu.make_async_copy(
            x_hbm.at[pl.ds(i * _BM, _BM), :], ibuf.at[s], isem.at[s]
        ).wait()

        @pl.when(i >= _NBUF)
        def _():
            pltpu.make_async_copy(
                obuf.at[s], o_hbm.at[pl.ds((i - _NBUF) * _BM, _BM), :],
                osem.at[s],
            ).wait()

        obuf[s] = _gelu(ibuf[s])
        pltpu.make_async_copy(
            obuf.at[s], o_hbm.at[pl.ds(i * _BM, _BM), :], osem.at[s]
        ).start()

        @pl.when(i + _NBUF < n)
        def _():
            pltpu.make_async_copy(
                x_hbm.at[pl.ds((i + _NBUF) * _BM, _BM), :], ibuf.at[s],
                isem.at[s],
            ).start()

        return carry

    lax.fori_loop(0, n, step, 0)

    for k in range(_NBUF):
        i = n - _NBUF + k
        s = i % _NBUF
        pltpu.make_async_copy(
            obuf.at[s], o_hbm.at[pl.ds(i * _BM, _BM), :], osem.at[s]
        ).wait()


def kernel(x, logit_decay, log_tau, log_blend):
    del logit_decay, log_tau, log_blend
    B, T, D = x.shape
    x2 = x.reshape(B * T, D)
    out = pl.pallas_call(
        _pipe_body,
        in_specs=[pl.BlockSpec(memory_space=pl.ANY)],
        out_specs=pl.BlockSpec(memory_space=pl.ANY),
        out_shape=jax.ShapeDtypeStruct(x2.shape, x2.dtype),
        scratch_shapes=[
            pltpu.VMEM((_NBUF, _BM, D), jnp.float32),
            pltpu.VMEM((_NBUF, _BM, D), jnp.float32),
            pltpu.SemaphoreType.DMA((_NBUF,)),
            pltpu.SemaphoreType.DMA((_NBUF,)),
        ],
    )(x2)
    return out.reshape(B, T, D)
